# baseline (speedup 1.0000x reference)
import functools

import jax
import jax.numpy as jnp
from jax import lax
from jax.experimental import pallas as pl
from jax.experimental.pallas import tpu as pltpu
from jax.experimental.pallas import tpu_sc as plsc

_NC = 2
_NS = 16
_NW = _NC * _NS
_L = 16


def _score_body(bpw, dims, h_hbm, r_hbm, t_hbm, ent_hbm, rel_hbm, bh_hbm,
                bt_hbm, out_hbm, ih_v, ir_v, it_v, rel_v, h_rows, t_rows,
                bh_v, bt_v, out_v, sem_rel, sem_h, sem_t, sem_bh, sem_bt):
  wid = lax.axis_index("s") * _NC + lax.axis_index("c")
  base = wid * bpw
  sl_all = pl.ds(base, bpw)

  cp_rel = pltpu.async_copy(rel_hbm, rel_v, sem_rel)

  pltpu.sync_copy(h_hbm.at[sl_all], ih_v)
  pltpu.sync_copy(t_hbm.at[sl_all], it_v)
  pltpu.sync_copy(r_hbm.at[sl_all], ir_v)

  cp_h = pltpu.async_copy(ent_hbm.at[ih_v], h_rows, sem_h)
  cp_t = pltpu.async_copy(ent_hbm.at[it_v], t_rows, sem_t)
  cp_bh = pltpu.async_copy(bh_hbm.at[ih_v], bh_v, sem_bh)
  cp_bt = pltpu.async_copy(bt_hbm.at[it_v], bt_v, sem_bt)
  cp_rel.wait()
  cp_h.wait()
  cp_t.wait()

  for j in range(bpw // _L):
    sl = pl.ds(_L * j, _L)
    rows = lax.iota(jnp.int32, _L) + (_L * j)
    irj = ir_v[sl]

    def dbody(d4, acc, rows=rows, irj=irj):
      for u in range(4):
        cd = jnp.full((_L,), d4 * 4 + u, jnp.int32)
        hv = plsc.load_gather(h_rows, [rows, cd])
        tv = plsc.load_gather(t_rows, [rows, cd])
        rv = plsc.load_gather(rel_v, [irj, cd])
        acc = acc + hv * rv * tv
      return acc

    acc = lax.fori_loop(0, dims // 4, dbody, jnp.zeros((_L,), jnp.float32))
    out_v[sl] = acc

  cp_bh.wait()
  cp_bt.wait()
  for j in range(bpw // _L):
    sl = pl.ds(_L * j, _L)
    out_v[sl] = out_v[sl] + bh_v[sl] + bt_v[sl]

  pltpu.sync_copy(out_v, out_hbm.at[sl_all])


def kernel(input_tensor, entities, relations, bias_head, bias_tail):
  batch = input_tensor.shape[0]
  dims = entities.shape[1]
  bpw = batch // _NW

  h_idx = input_tensor[:, 0]
  r_idx = input_tensor[:, 1]
  t_idx = input_tensor[:, 2]
  bh_flat = bias_head.reshape(-1)
  bt_flat = bias_tail.reshape(-1)

  mesh = plsc.VectorSubcoreMesh(
      core_axis_name="c", subcore_axis_name="s",
      num_cores=_NC, num_subcores=_NS)

  fn = pl.kernel(
      functools.partial(_score_body, bpw, dims),
      out_type=jax.ShapeDtypeStruct((batch,), jnp.float32),
      mesh=mesh,
      compiler_params=pltpu.CompilerParams(
          needs_layout_passes=False, use_tc_tiling_on_sc=False),
      scratch_types=[
          pltpu.VMEM((bpw,), jnp.int32),
          pltpu.VMEM((bpw,), jnp.int32),
          pltpu.VMEM((bpw,), jnp.int32),
          pltpu.VMEM(relations.shape, jnp.float32),
          pltpu.VMEM((bpw, dims), jnp.float32),
          pltpu.VMEM((bpw, dims), jnp.float32),
          pltpu.VMEM((bpw,), jnp.float32),
          pltpu.VMEM((bpw,), jnp.float32),
          pltpu.VMEM((bpw,), jnp.float32),
          pltpu.SemaphoreType.DMA,
          pltpu.SemaphoreType.DMA,
          pltpu.SemaphoreType.DMA,
          pltpu.SemaphoreType.DMA,
          pltpu.SemaphoreType.DMA,
      ],
  )
  score = fn(h_idx, r_idx, t_idx, entities, relations, bh_flat, bt_flat)
  return score.reshape(batch, 1)


# speedup vs baseline: 2.9697x; 2.9697x over previous
import functools

import jax
import jax.numpy as jnp
from jax import lax
from jax.experimental import pallas as pl
from jax.experimental.pallas import tpu as pltpu
from jax.experimental.pallas import tpu_sc as plsc

_NC = 2
_NS = 16
_NW = _NC * _NS
_L = 16
_NBUF = 4


def _extract(vec, lane, k):
  return jnp.sum(jnp.where(lane == k, vec, 0))


def _score_body(bpw, dims, ne, edge,
                h_hbm, r_hbm, t_hbm, entT_hbm, tail_hbm, rel_hbm, bh_hbm,
                bt_hbm, out_hbm,
                ih_v, ir_v, it_v, rel_v, hb0, hb1, hb2, hb3, tb0, tb1, tb2,
                tb3, bh_v, bt_v, out_v,
                s0, s1, s2, s3, sem_bh, sem_bt):
  hbufs = (hb0, hb1, hb2, hb3)
  tbufs = (tb0, tb1, tb2, tb3)
  sems = (s0, s1, s2, s3)
  ngrp = bpw // _NBUF

  wid = lax.axis_index("s") * _NC + lax.axis_index("c")
  base = wid * bpw
  sl_all = pl.ds(base, bpw)
  lane = lax.iota(jnp.int32, _L)

  pltpu.sync_copy(h_hbm.at[sl_all], ih_v)
  pltpu.sync_copy(t_hbm.at[sl_all], it_v)
  pltpu.sync_copy(r_hbm.at[sl_all], ir_v)
  cp_bh = pltpu.async_copy(bh_hbm.at[ih_v], bh_v, sem_bh)
  cp_bt = pltpu.async_copy(bt_hbm.at[it_v], bt_v, sem_bt)
  pltpu.sync_copy(rel_hbm, rel_v)

  def fire(k, b):
    blk = (k // _L) * _L
    lk = k - blk
    ids_h = ih_v[pl.ds(blk, _L)]
    ids_t = it_v[pl.ds(blk, _L)]
    idh = _extract(ids_h, lane, lk)
    idt = _extract(ids_t, lane, lk)

    @pl.when(idh < edge)
    def _():
      st = pl.multiple_of((idh // 128) * 128, 128)
      pltpu.async_copy(entT_hbm.at[:, pl.ds(st, 128)], hbufs[b], sems[b])

    @pl.when(idh >= edge)
    def _():
      pltpu.async_copy(tail_hbm, hbufs[b], sems[b])

    @pl.when(idt < edge)
    def _():
      st = pl.multiple_of((idt // 128) * 128, 128)
      pltpu.async_copy(entT_hbm.at[:, pl.ds(st, 128)], tbufs[b], sems[b])

    @pl.when(idt >= edge)
    def _():
      pltpu.async_copy(tail_hbm, tbufs[b], sems[b])

  for b in range(_NBUF):
    fire(b, b)

  def group(g, carry):
    for b in range(_NBUF):
      k = g * _NBUF + b
      pltpu.make_async_copy(entT_hbm.at[:, pl.ds(0, 128)], hbufs[b],
                            sems[b]).wait()
      pltpu.make_async_copy(entT_hbm.at[:, pl.ds(0, 128)], tbufs[b],
                            sems[b]).wait()

      blk = (k // _L) * _L
      lk = k - blk
      sl = pl.ds(blk, _L)
      idh = _extract(ih_v[sl], lane, lk)
      idt = _extract(it_v[sl], lane, lk)
      rk = _extract(ir_v[sl], lane, lk)
      ich = jnp.where(idh >= edge, idh - edge, idh - (idh // 128) * 128)
      ict = jnp.where(idt >= edge, idt - edge, idt - (idt // 128) * 128)
      ichv = jnp.full((_L,), ich, jnp.int32)
      ictv = jnp.full((_L,), ict, jnp.int32)
      rkv = jnp.full((_L,), rk, jnp.int32)

      acc = jnp.zeros((_L,), jnp.float32)
      for c in range(dims // _L):
        cd = lane + (_L * c)
        hv = plsc.load_gather(hbufs[b], [cd, ichv])
        tv = plsc.load_gather(tbufs[b], [cd, ictv])
        rv = plsc.load_gather(rel_v, [rkv, cd])
        acc = acc + hv * rv * tv
      s = jnp.sum(acc)
      out_v[sl] = jnp.where(lane == lk, s, out_v[sl])

      @pl.when(g + 1 < ngrp)
      def _():
        fire(k + _NBUF, b)
    return carry

  lax.fori_loop(0, ngrp, group, 0)

  cp_bh.wait()
  cp_bt.wait()
  for j in range(bpw // _L):
    sl = pl.ds(_L * j, _L)
    out_v[sl] = out_v[sl] + bh_v[sl] + bt_v[sl]

  pltpu.sync_copy(out_v, out_hbm.at[sl_all])


def kernel(input_tensor, entities, relations, bias_head, bias_tail):
  batch = input_tensor.shape[0]
  ne, dims = entities.shape
  bpw = batch // _NW
  edge = (ne // 128) * 128

  h_idx = input_tensor[:, 0]
  r_idx = input_tensor[:, 1]
  t_idx = input_tensor[:, 2]
  bh_flat = bias_head.reshape(-1)
  bt_flat = bias_tail.reshape(-1)
  ent_t = entities.T
  tail = jnp.pad(entities[edge:].T, ((0, 0), (0, 128 - (ne - edge))))

  mesh = plsc.VectorSubcoreMesh(
      core_axis_name="c", subcore_axis_name="s",
      num_cores=_NC, num_subcores=_NS)

  fn = pl.kernel(
      functools.partial(_score_body, bpw, dims, ne, edge),
      out_type=jax.ShapeDtypeStruct((batch,), jnp.float32),
      mesh=mesh,
      compiler_params=pltpu.CompilerParams(
          needs_layout_passes=False, use_tc_tiling_on_sc=True),
      scratch_types=[
          pltpu.VMEM((bpw,), jnp.int32),
          pltpu.VMEM((bpw,), jnp.int32),
          pltpu.VMEM((bpw,), jnp.int32),
          pltpu.VMEM(relations.shape, jnp.float32),
          pltpu.VMEM((dims, 128), jnp.float32),
          pltpu.VMEM((dims, 128), jnp.float32),
          pltpu.VMEM((dims, 128), jnp.float32),
          pltpu.VMEM((dims, 128), jnp.float32),
          pltpu.VMEM((dims, 128), jnp.float32),
          pltpu.VMEM((dims, 128), jnp.float32),
          pltpu.VMEM((dims, 128), jnp.float32),
          pltpu.VMEM((dims, 128), jnp.float32),
          pltpu.VMEM((bpw,), jnp.float32),
          pltpu.VMEM((bpw,), jnp.float32),
          pltpu.VMEM((bpw,), jnp.float32),
          pltpu.SemaphoreType.DMA,
          pltpu.SemaphoreType.DMA,
          pltpu.SemaphoreType.DMA,
          pltpu.SemaphoreType.DMA,
          pltpu.SemaphoreType.DMA,
          pltpu.SemaphoreType.DMA,
      ],
  )
  score = fn(h_idx, r_idx, t_idx, ent_t, tail, relations, bh_flat, bt_flat)
  return score.reshape(batch, 1)
